# baseline (speedup 1.0000x reference)
import jax
import jax.numpy as jnp
from jax import lax
from jax.experimental import pallas as pl
from jax.experimental.pallas import tpu as pltpu
from jax.experimental.pallas import tpu_sc as plsc

NUM_FIELDS = 26
VOCAB = 100000
EMB_DIM = 32
BATCH = 16384

NC = 2
NS = 16
L = 16
NW = NC * NS

NFP = 32
JB = BATCH // 128
JPW = JB // NW
CPT = NUM_FIELDS * JPW
FB = NUM_FIELDS * EMB_DIM // 8


def _transpose_chunk(gbuf, tbuf, iotas):
    @pl.loop(0, EMB_DIM)
    def _per_feat(c):
        cr = c // 8
        fs = lax.rem(c, 8)
        col = jnp.full((L,), 0, jnp.int32) + c
        for b8 in range(8):
            v = plsc.load_gather(gbuf, [iotas[b8], col])
            tbuf[cr, 0, fs, pl.ds(b8 * L, L)] = v


def _body(inp4, table_hbm, out4, idxs, gbufs, tbufs, gsem0, gsem1, wsem):
    wid = lax.axis_index("s") * NC + lax.axis_index("c")
    j0 = wid * JPW

    pltpu.sync_copy(inp4.at[:, pl.ds(j0, JPW)], idxs)

    iotas = [lax.iota(jnp.int32, L) + (b8 * L) for b8 in range(8)]
    gsems = (gsem0, gsem1)

    def add_offset_and_fire(t, s):
        field = t // JPW
        jj = lax.rem(t, JPW)
        fr = field // 8
        fs = lax.rem(field, 8)
        for b8 in range(8):
            sl = (fr, jj, fs, pl.ds(b8 * L, L))
            idxs[sl] = idxs[sl] + field * VOCAB
        pltpu.async_copy(
            table_hbm.at[idxs.at[fr, jj, fs]], gbufs.at[s], gsems[s])

    def wait_gather(s):
        pltpu.make_async_copy(
            table_hbm.at[pl.ds(0, 128)], gbufs.at[s], gsems[s]).wait()

    def fire_wb(t, s):
        field = t // JPW
        jj = lax.rem(t, JPW)
        pltpu.async_copy(
            tbufs.at[s],
            out4.at[pl.ds(4 * field, 4), pl.ds(j0 + jj, 1)], wsem)

    def drain_wb(s):
        pltpu.make_async_copy(
            tbufs.at[s], out4.at[pl.ds(0, 4), pl.ds(0, 1)], wsem).wait()

    add_offset_and_fire(0, 0)
    add_offset_and_fire(1, 1)
    for s in range(2):
        t = s
        wait_gather(s)
        _transpose_chunk(gbufs.at[s], tbufs.at[s], iotas)
        fire_wb(t, s)
        add_offset_and_fire(t + 2, s)

    @pl.loop(1, CPT // 2 - 1)
    def _main(k):
        for s in range(2):
            t = 2 * k + s
            wait_gather(s)
            drain_wb(s)
            _transpose_chunk(gbufs.at[s], tbufs.at[s], iotas)
            fire_wb(t, s)
            add_offset_and_fire(t + 2, s)

    for s in range(2):
        t = CPT - 2 + s
        wait_gather(s)
        drain_wb(s)
        _transpose_chunk(gbufs.at[s], tbufs.at[s], iotas)
        fire_wb(t, s)
    for s in range(2):
        drain_wb(s)


def _sc_gather():
    mesh = plsc.VectorSubcoreMesh(
        core_axis_name="c", subcore_axis_name="s",
        num_cores=NC, num_subcores=NS)
    return pl.kernel(
        _body,
        out_type=jax.ShapeDtypeStruct((FB, JB, 8, 128), jnp.float32),
        mesh=mesh,
        scratch_types=[
            pltpu.VMEM((NFP // 8, JPW, 8, 128), jnp.int32),
            pltpu.VMEM((2, 128, EMB_DIM), jnp.float32),
            pltpu.VMEM((2, 4, 1, 8, 128), jnp.float32),
            pltpu.SemaphoreType.DMA,
            pltpu.SemaphoreType.DMA,
            pltpu.SemaphoreType.DMA,
        ],
        compiler_params=pltpu.CompilerParams(
            use_tc_tiling_on_sc=False, needs_layout_passes=False),
    )


def kernel(inputs, tables):
    inp_p = jnp.pad(inputs.astype(jnp.int32), ((0, 0), (0, NFP - NUM_FIELDS)))
    inp4 = inp_p.T.reshape(NFP // 8, 8, JB, 128).transpose(0, 2, 1, 3)
    tables_flat = tables.reshape(NUM_FIELDS * VOCAB, EMB_DIM)
    out4 = _sc_gather()(inp4, tables_flat)
    return out4.transpose(1, 3, 0, 2).reshape(BATCH, NUM_FIELDS * EMB_DIM)


# speedup vs baseline: 1.0351x; 1.0351x over previous
import jax
import jax.numpy as jnp
from jax import lax
from jax.experimental import pallas as pl
from jax.experimental.pallas import tpu as pltpu
from jax.experimental.pallas import tpu_sc as plsc

NUM_FIELDS = 26
VOCAB = 100000
EMB_DIM = 32
BATCH = 16384

NC = 2
NS = 16
L = 16
NW = NC * NS

NFP = 32
JB = BATCH // 128
JPW = JB // NW
CPT = NUM_FIELDS * JPW


def _body(inp4, table_hbm, out3, idxs, gbufs, gsem0, gsem1, wsem):
    wid = lax.axis_index("s") * NC + lax.axis_index("c")
    j0 = wid * JPW

    pltpu.sync_copy(inp4.at[:, pl.ds(j0, JPW)], idxs)

    gsems = (gsem0, gsem1)

    def fire_gather(t, s):
        field = t // JPW
        jj = lax.rem(t, JPW)
        fr = field // 8
        fs = lax.rem(field, 8)
        pltpu.async_copy(
            table_hbm.at[field].at[idxs.at[fr, jj, fs]],
            gbufs.at[s], gsems[s])

    def wait_gather(s):
        pltpu.make_async_copy(
            table_hbm.at[0].at[pl.ds(0, 128)], gbufs.at[s], gsems[s]).wait()

    def fire_wb(t, s):
        field = t // JPW
        jj = lax.rem(t, JPW)
        pltpu.async_copy(
            gbufs.at[s, :, pl.ds(0, EMB_DIM)],
            out3.at[field].at[pl.ds((j0 + jj) * 128, 128)], wsem)

    def drain_wb(s):
        pltpu.make_async_copy(
            gbufs.at[s, :, pl.ds(0, EMB_DIM)],
            out3.at[0].at[pl.ds(0, 128)], wsem).wait()

    fire_gather(0, 0)
    fire_gather(1, 1)

    @pl.loop(0, CPT // 2 - 1)
    def _main(k):
        for s in range(2):
            t = 2 * k + s
            wait_gather(s)
            fire_wb(t, s)
            drain_wb(s)
            fire_gather(t + 2, s)

    for s in range(2):
        t = CPT - 2 + s
        wait_gather(s)
        fire_wb(t, s)
        drain_wb(s)


def _sc_gather():
    mesh = plsc.VectorSubcoreMesh(
        core_axis_name="c", subcore_axis_name="s",
        num_cores=NC, num_subcores=NS)
    return pl.kernel(
        _body,
        out_type=jax.ShapeDtypeStruct((NUM_FIELDS, BATCH, EMB_DIM),
                                      jnp.float32),
        mesh=mesh,
        scratch_types=[
            pltpu.VMEM((NFP // 8, JPW, 8, 128), jnp.int32),
            pltpu.VMEM((2, 128, 128), jnp.float32),
            pltpu.SemaphoreType.DMA,
            pltpu.SemaphoreType.DMA,
            pltpu.SemaphoreType.DMA,
        ],
        compiler_params=pltpu.CompilerParams(
            use_tc_tiling_on_sc=False, needs_layout_passes=False),
    )


def kernel(inputs, tables):
    inp_p = jnp.pad(inputs.astype(jnp.int32), ((0, 0), (0, NFP - NUM_FIELDS)))
    inp4 = inp_p.T.reshape(NFP // 8, 8, JB, 128).transpose(0, 2, 1, 3)
    tables_p = jnp.pad(tables, ((0, 0), (0, 0), (0, 128 - EMB_DIM)))
    out3 = _sc_gather()(inp4, tables_p)
    return out3.transpose(1, 0, 2).reshape(BATCH, NUM_FIELDS * EMB_DIM)


# speedup vs baseline: 1.0967x; 1.0595x over previous
import jax
import jax.numpy as jnp
from jax import lax
from jax.experimental import pallas as pl
from jax.experimental.pallas import tpu as pltpu
from jax.experimental.pallas import tpu_sc as plsc

NUM_FIELDS = 26
VOCAB = 100000
EMB_DIM = 32
BATCH = 16384

NC = 2
NS = 16
L = 16
NW = NC * NS

NFP = 32
JB = BATCH // 128
JPW = JB // NW
NG = NUM_FIELDS
PAD_RATIO = 128 // EMB_DIM


def _body(inp4, table_hbm, out3, idxs, bufs, gsem0, gsem1, wsem0, wsem1):
    wid = lax.axis_index("s") * NC + lax.axis_index("c")
    j0 = wid * JPW

    pltpu.sync_copy(inp4.at[:, pl.ds(j0, JPW)], idxs)

    @pl.loop(0, NUM_FIELDS * JPW)
    def _scale(r):
        field = r // JPW
        jj = lax.rem(r, JPW)
        fr = field // 8
        fs = lax.rem(field, 8)
        for c8 in range(8):
            sl = (fr, jj, fs, pl.ds(c8 * L, L))
            idxs[sl] = idxs[sl] * PAD_RATIO

    gsems = (gsem0, gsem1)
    wsems = (wsem0, wsem1)

    def fire_gathers(f, s):
        fr = f // 8
        fs = lax.rem(f, 8)
        for b in range(JPW):
            pltpu.async_copy(
                table_hbm.at[f].at[idxs.at[fr, b, fs]],
                bufs.at[pl.ds((s * JPW + b) * 128, 128)], gsems[s])

    def drain_gathers(s):
        pltpu.make_async_copy(
            table_hbm.at[0].at[pl.ds(0, JPW * 128)],
            bufs.at[pl.ds(s * JPW * 128, JPW * 128)], gsems[s]).wait()

    def fire_wbs(f, s):
        for b in range(JPW):
            pltpu.async_copy(
                bufs.at[pl.ds((s * JPW + b) * 128, 128)],
                out3.at[f].at[pl.ds((j0 + b) * 128, 128)], wsems[s])

    def drain_wbs(s):
        pltpu.make_async_copy(
            bufs.at[pl.ds(s * JPW * 128, JPW * 128)],
            out3.at[0].at[pl.ds(0, JPW * 128)], wsems[s]).wait()

    fire_gathers(0, 0)
    fire_gathers(1, 1)

    @pl.loop(0, NG // 2 - 1)
    def _main(k):
        for s in range(2):
            f = 2 * k + s
            drain_gathers(s)
            fire_wbs(f, s)
            drain_wbs(s)
            fire_gathers(f + 2, s)

    for s in range(2):
        f = NG - 2 + s
        drain_gathers(s)
        fire_wbs(f, s)
        drain_wbs(s)


def _sc_gather():
    mesh = plsc.VectorSubcoreMesh(
        core_axis_name="c", subcore_axis_name="s",
        num_cores=NC, num_subcores=NS)
    return pl.kernel(
        _body,
        out_type=jax.ShapeDtypeStruct((NUM_FIELDS, BATCH, EMB_DIM),
                                      jnp.float32),
        mesh=mesh,
        scratch_types=[
            pltpu.VMEM((NFP // 8, JPW, 8, 128), jnp.int32),
            pltpu.VMEM((2 * JPW * 128, EMB_DIM), jnp.float32),
            pltpu.SemaphoreType.DMA,
            pltpu.SemaphoreType.DMA,
            pltpu.SemaphoreType.DMA,
            pltpu.SemaphoreType.DMA,
        ],
        compiler_params=pltpu.CompilerParams(
            use_tc_tiling_on_sc=False, needs_layout_passes=False),
    )


def kernel(inputs, tables):
    inp_p = jnp.pad(inputs.astype(jnp.int32), ((0, 0), (0, NFP - NUM_FIELDS)))
    inp4 = inp_p.T.reshape(NFP // 8, 8, JB, 128).transpose(0, 2, 1, 3)
    tables_p = jnp.pad(tables, ((0, 0), (0, 0), (0, 128 - EMB_DIM)))
    tables_v = tables_p.reshape(NUM_FIELDS, PAD_RATIO * VOCAB, EMB_DIM)
    out3 = _sc_gather()(inp4, tables_v)
    return out3.transpose(1, 0, 2).reshape(BATCH, NUM_FIELDS * EMB_DIM)


# speedup vs baseline: 1.1763x; 1.0726x over previous
import jax
import jax.numpy as jnp
from jax import lax
from jax.experimental import pallas as pl
from jax.experimental.pallas import tpu as pltpu
from jax.experimental.pallas import tpu_sc as plsc

NUM_FIELDS = 26
VOCAB = 100000
EMB_DIM = 32
BATCH = 16384

NC = 2
NS = 16
L = 16
NW = NC * NS

R = BATCH * NUM_FIELDS
CHUNK = 128
CPW = R // (NW * CHUNK)
NB = 4
NG = CPW // NB


def _body(inp_hbm, table_hbm, out_hbm, idx_v, bufs, gsem0, gsem1, wsem0, wsem1):
    wid = lax.axis_index("s") * NC + lax.axis_index("c")
    row0 = wid * CPW

    pltpu.sync_copy(inp_hbm.at[pl.ds(row0, CPW)], idx_v)

    @pl.loop(0, CPW)
    def _add_offsets(r):
        for c in range(CHUNK // L):
            base = r * CHUNK + c * L
            field = lax.rem(lax.iota(jnp.int32, L) + base, NUM_FIELDS)
            sl = (r, pl.ds(c * L, L))
            idx_v[sl] = idx_v[sl] * 4 + field * (4 * VOCAB)

    gsems = (gsem0, gsem1)
    wsems = (wsem0, wsem1)

    def fire_gathers(g, s):
        for b in range(NB):
            pltpu.async_copy(
                table_hbm.at[idx_v.at[g * NB + b]],
                bufs.at[pl.ds((s * NB + b) * CHUNK, CHUNK)],
                gsems[s])

    def drain_gathers(s):
        pltpu.make_async_copy(
            out_hbm.at[pl.ds(0, NB * CHUNK)],
            bufs.at[pl.ds(s * NB * CHUNK, NB * CHUNK)],
            gsems[s]).wait()

    def fire_wbs(g, s):
        for b in range(NB):
            dst_row = (row0 + g * NB + b) * CHUNK
            pltpu.async_copy(
                bufs.at[pl.ds((s * NB + b) * CHUNK, CHUNK)],
                out_hbm.at[pl.ds(dst_row, CHUNK)],
                wsems[s])

    def drain_wbs(s):
        pltpu.make_async_copy(
            bufs.at[pl.ds(s * NB * CHUNK, NB * CHUNK)],
            out_hbm.at[pl.ds(0, NB * CHUNK)],
            wsems[s]).wait()

    fire_gathers(0, 0)
    fire_gathers(1, 1)

    @pl.loop(0, NG // 2 - 1)
    def _main(k):
        for s in range(2):
            g = 2 * k + s
            drain_gathers(s)
            fire_wbs(g, s)
            drain_wbs(s)
            fire_gathers(g + 2, s)

    for s in range(2):
        g = NG - 2 + s
        drain_gathers(s)
        fire_wbs(g, s)
        drain_wbs(s)


def _sc_gather():
    mesh = plsc.VectorSubcoreMesh(
        core_axis_name="c", subcore_axis_name="s",
        num_cores=NC, num_subcores=NS)
    return pl.kernel(
        _body,
        out_type=jax.ShapeDtypeStruct((R, EMB_DIM), jnp.float32),
        mesh=mesh,
        scratch_types=[
            pltpu.VMEM((CPW, CHUNK), jnp.int32),
            pltpu.VMEM((2 * NB * CHUNK, EMB_DIM), jnp.float32),
            pltpu.SemaphoreType.DMA,
            pltpu.SemaphoreType.DMA,
            pltpu.SemaphoreType.DMA,
            pltpu.SemaphoreType.DMA,
        ],
        compiler_params=pltpu.CompilerParams(use_tc_tiling_on_sc=False),
    )


def kernel(inputs, tables):
    inp2 = inputs.astype(jnp.int32).reshape(R // CHUNK, CHUNK)
    tables_pad = jnp.pad(tables, ((0, 0), (0, 0), (0, 128 - EMB_DIM)))
    tables_flat = tables_pad.reshape(NUM_FIELDS * 4 * VOCAB, EMB_DIM)
    out = _sc_gather()(inp2, tables_flat)
    return out.reshape(BATCH, NUM_FIELDS * EMB_DIM)
